# Initial kernel scaffold; baseline (speedup 1.0000x reference)
import jax
import jax.numpy as jnp
from jax.experimental import pallas as pl


def kernel(hidden, outputs, attention_w, attention_b, attention_v):
    raise NotImplementedError("write your pallas kernel here")


# speedup vs baseline: 1.4778x; 1.4778x over previous
import functools

import jax
import jax.numpy as jnp
from jax.experimental import pallas as pl
from jax.experimental.pallas import tpu as pltpu


def _score_kernel(hproj_ref, w_ref, v_ref, o_ref, out_ref):
    o = o_ref[...]
    st, b, h = o.shape
    lhs = o.reshape(st * b, h).astype(jnp.bfloat16)
    oproj = jnp.dot(lhs, w_ref[...],
                    preferred_element_type=jnp.float32).reshape(st, b, h)
    t = jnp.tanh(oproj + hproj_ref[...][None, :, :])
    out_ref[...] = jnp.sum(t * v_ref[...][None, :, :], axis=2)


def _softmax_t_kernel(s_ref, out_ref):
    s = s_ref[...]
    m = jnp.max(s, axis=0, keepdims=True)
    e = jnp.exp(s - m)
    p = e * (1.0 / jnp.sum(e, axis=0, keepdims=True))
    out_ref[...] = jnp.transpose(p)[:, None, :]


def _luong_concat(hidden, outputs, w, b, v, *, interpret=False):
    S, B, H = outputs.shape
    hp = jax.lax.Precision.HIGHEST

    hidden_bm = hidden.reshape(B, H).astype(jnp.float32)
    w = w.astype(jnp.float32)
    hproj = jnp.dot(hidden_bm, jnp.transpose(w[:, :H]), precision=hp) + b[None, :]
    w_o_t = jnp.transpose(w[:, H:]).astype(jnp.bfloat16)
    v2 = v.astype(jnp.float32).reshape(1, H)

    st = 64
    n_tiles = pl.cdiv(S, st)

    def rep(shape):
        return pl.BlockSpec(shape, lambda s: (0,) * len(shape))

    flops = 2 * S * B * H * H
    cost = pl.CostEstimate(flops=flops, transcendentals=S * B * H,
                           bytes_accessed=S * B * H * 4 + S * B * 4)

    scores = pl.pallas_call(
        _score_kernel,
        out_shape=jax.ShapeDtypeStruct((S, B), jnp.float32),
        grid=(n_tiles,),
        in_specs=[rep((B, H)), rep((H, H)), rep((1, H)),
                  pl.BlockSpec((st, B, H), lambda s: (s, 0, 0))],
        out_specs=pl.BlockSpec((st, B), lambda s: (s, 0)),
        compiler_params=pltpu.CompilerParams(
            dimension_semantics=("parallel",),
            vmem_limit_bytes=56 * 1024 * 1024),
        cost_estimate=cost,
        interpret=interpret,
    )(hproj, w_o_t, v2, outputs)

    vmem = pl.BlockSpec(memory_space=pltpu.MemorySpace.VMEM)
    return pl.pallas_call(
        _softmax_t_kernel,
        out_shape=jax.ShapeDtypeStruct((B, 1, S), jnp.float32),
        in_specs=[vmem],
        out_specs=vmem,
        interpret=interpret,
    )(scores)


def kernel(hidden, outputs, attention_w, attention_b, attention_v):
    return _luong_concat(hidden, outputs, attention_w, attention_b,
                         attention_v)


# speedup vs baseline: 1.6662x; 1.1275x over previous
import functools

import jax
import jax.numpy as jnp
from jax.experimental import pallas as pl
from jax.experimental.pallas import tpu as pltpu


def _score_kernel(hproj_ref, w_ref, v_ref, o_ref, out_ref, *, chunk_s):
    st = o_ref.shape[0]
    w = w_ref[...]
    hp = hproj_ref[...][None, :, :]
    vv = v_ref[...][None, :, :]
    for c in range(st // chunk_s):
        o = o_ref[pl.ds(c * chunk_s, chunk_s), :, :]
        cs, b, h = o.shape
        lhs = o.reshape(cs * b, h).astype(jnp.bfloat16)
        oproj = jnp.dot(lhs, w,
                        preferred_element_type=jnp.float32).reshape(cs, b, h)
        t = jnp.tanh(oproj + hp)
        out_ref[pl.ds(c * chunk_s, chunk_s), :] = jnp.sum(t * vv, axis=2)


def _softmax_t_kernel(s_ref, out_ref):
    s = s_ref[...]
    m = jnp.max(s, axis=0, keepdims=True)
    e = jnp.exp(s - m)
    p = e * (1.0 / jnp.sum(e, axis=0, keepdims=True))
    out_ref[...] = jnp.transpose(p)[:, None, :]


def _luong_concat(hidden, outputs, w, b, v, *, interpret=False):
    S, B, H = outputs.shape
    hp = jax.lax.Precision.HIGHEST

    hidden_bm = hidden.reshape(B, H).astype(jnp.float32)
    w = w.astype(jnp.float32)
    hproj = jnp.dot(hidden_bm, jnp.transpose(w[:, :H]), precision=hp) + b[None, :]
    w_o_t = jnp.transpose(w[:, H:]).astype(jnp.bfloat16)
    v2 = v.astype(jnp.float32).reshape(1, H)

    st = 128
    chunk_s = 16
    n_tiles = pl.cdiv(S, st)

    def rep(shape):
        return pl.BlockSpec(shape, lambda s: (0,) * len(shape))

    flops = 2 * S * B * H * H
    cost = pl.CostEstimate(flops=flops, transcendentals=S * B * H,
                           bytes_accessed=S * B * H * 4 + S * B * 4)

    scores = pl.pallas_call(
        functools.partial(_score_kernel, chunk_s=chunk_s),
        out_shape=jax.ShapeDtypeStruct((S, B), jnp.float32),
        grid=(n_tiles,),
        in_specs=[rep((B, H)), rep((H, H)), rep((1, H)),
                  pl.BlockSpec((st, B, H), lambda s: (s, 0, 0))],
        out_specs=pl.BlockSpec((st, B), lambda s: (s, 0)),
        compiler_params=pltpu.CompilerParams(
            dimension_semantics=("parallel",),
            vmem_limit_bytes=60 * 1024 * 1024),
        cost_estimate=cost,
        interpret=interpret,
    )(hproj, w_o_t, v2, outputs)

    vmem = pl.BlockSpec(memory_space=pltpu.MemorySpace.VMEM)
    return pl.pallas_call(
        _softmax_t_kernel,
        out_shape=jax.ShapeDtypeStruct((B, 1, S), jnp.float32),
        in_specs=[vmem],
        out_specs=vmem,
        interpret=interpret,
    )(scores)


def kernel(hidden, outputs, attention_w, attention_b, attention_v):
    return _luong_concat(hidden, outputs, attention_w, attention_b,
                         attention_v)
